# baseline (speedup 1.0000x reference)
import functools

import jax
import jax.numpy as jnp
from jax.experimental import pallas as pl
from jax.experimental.pallas import tpu as pltpu
from jax.experimental.pallas import tpu_sc as plsc

_EMBED = 1024
_HIDDEN = 512
_E = 64
_K = 8
_T = 2048
_BM = 256
_ROWS = _T * _K
_MAX_BLOCKS = _ROWS // _BM + _E
_NW = 32
_GCH = 64
_CHT = 8


def _sc_mesh():
    return plsc.VectorSubcoreMesh(core_axis_name="c", subcore_axis_name="s")


def _sc_gather_rows(table, idx, n_rows):
    per_w = n_rows // _NW
    nch = per_w // _GCH

    @functools.partial(
        pl.kernel,
        out_type=jax.ShapeDtypeStruct((n_rows, _EMBED), jnp.float32),
        mesh=_sc_mesh(),
        scratch_types=[
            pltpu.VMEM((_GCH,), jnp.int32),
            pltpu.VMEM((_GCH, _EMBED), jnp.float32),
            pltpu.SemaphoreType.DMA,
        ],
    )
    def k(tab_hbm, idx_hbm, out_hbm, idx_v, rows_v, sem):
        wid = jax.lax.axis_index("s") * 2 + jax.lax.axis_index("c")
        base = wid * per_w

        @pl.loop(0, nch)
        def _(c):
            b = base + c * _GCH
            pltpu.sync_copy(idx_hbm.at[pl.ds(b, _GCH)], idx_v)
            pltpu.async_copy(tab_hbm.at[idx_v], rows_v, sem).wait()
            pltpu.sync_copy(rows_v, out_hbm.at[pl.ds(b, _GCH)])

    return k(table, idx)


def _sum8_kernel(x_ref, o_ref):
    o_ref[...] = x_ref[...].reshape(_BM, _K, _EMBED).sum(axis=1)


def _sum8(tm):
    return pl.pallas_call(
        _sum8_kernel,
        grid=(_T // _BM,),
        in_specs=[pl.BlockSpec((_BM * _K, _EMBED), lambda i: (i, 0))],
        out_specs=pl.BlockSpec((_BM, _EMBED), lambda i: (i, 0)),
        out_shape=jax.ShapeDtypeStruct((_T, _EMBED), jnp.float32),
    )(tm)


def _ffn_block_kernel(be_ref, xm_ref, vd_ref, x_ref, ge_ref, lin_ref, w_ref, out_ref):
    i = pl.program_id(0)

    @pl.when(vd_ref[i] == 1)
    def _():
        xb = x_ref[...].astype(jnp.bfloat16)
        g0 = ge_ref[0, 0].astype(jnp.bfloat16)
        g1 = ge_ref[0, 1].astype(jnp.bfloat16)
        x1 = jax.lax.dot_general(xb, g0, (((1,), (1,)), ((), ())),
                                 preferred_element_type=jnp.float32)
        x2 = jax.lax.dot_general(xb, g1, (((1,), (1,)), ((), ())),
                                 preferred_element_type=jnp.float32)
        act = (jax.nn.gelu(x1) * x2).astype(jnp.bfloat16)
        lin = lin_ref[0].astype(jnp.bfloat16)
        out = jax.lax.dot_general(act, lin, (((1,), (0,)), ((), ())),
                                  preferred_element_type=jnp.float32)
        out_ref[...] = out * jnp.transpose(w_ref[0])


def _grouped_ffn(sorted_x, gating_einsum, linear, wrow, block_expert, block_xmap, block_valid):
    grid_spec = pltpu.PrefetchScalarGridSpec(
        num_scalar_prefetch=3,
        grid=(_MAX_BLOCKS,),
        in_specs=[
            pl.BlockSpec((_BM, _EMBED), lambda i, be, xm, vd: (xm[i], 0)),
            pl.BlockSpec((1, 2, _HIDDEN, _EMBED), lambda i, be, xm, vd: (be[i], 0, 0, 0)),
            pl.BlockSpec((1, _HIDDEN, _EMBED), lambda i, be, xm, vd: (be[i], 0, 0)),
            pl.BlockSpec((1, 1, _BM), lambda i, be, xm, vd: (xm[i], 0, 0)),
        ],
        out_specs=pl.BlockSpec((_BM, _EMBED), lambda i, be, xm, vd: (xm[i], 0)),
    )
    return pl.pallas_call(
        _ffn_block_kernel,
        grid_spec=grid_spec,
        out_shape=jax.ShapeDtypeStruct((_MAX_BLOCKS * _BM, _EMBED), jnp.float32),
        compiler_params=pltpu.CompilerParams(
            dimension_semantics=("parallel",)),
    )(block_expert, block_xmap, block_valid, sorted_x, gating_einsum, linear, wrow)


def kernel(x, router_logits, gating_einsum, linear, per_expert_scale, router_scale):
    g, s, d = x.shape
    t = g * s
    xf = x.reshape(t, d)

    var = jnp.mean(jnp.square(xf), axis=-1, keepdims=True)
    ri = xf * jax.lax.rsqrt(var + 1e-06)
    ri = ri * jax.lax.rsqrt(jnp.float32(d)) * router_scale
    logits = ri @ router_logits
    probs = jax.nn.softmax(logits, axis=-1)
    _, choices = jax.lax.approx_max_k(logits, k=_K)
    indicator = jax.nn.one_hot(choices, _E, dtype=probs.dtype).sum(axis=-2)
    renorm = jnp.sum(indicator * probs, axis=-1, keepdims=True)
    renorm = jnp.where(renorm > 0.0, renorm, 1.0)
    cw = jnp.take_along_axis(probs / renorm, choices, axis=-1)
    cw = cw * per_expert_scale[choices]

    cf = choices.reshape(-1)
    nch = 128
    cfm = cf.reshape(nch, _ROWS // nch)
    ohc = (cfm[:, :, None] == jnp.arange(_E)).astype(jnp.bfloat16)
    tril = jnp.tril(jnp.ones((_ROWS // nch, _ROWS // nch), jnp.bfloat16))
    within = jnp.einsum('ij,cje->cie', tril, ohc,
                        preferred_element_type=jnp.float32)
    counts_chunk = within[:, -1, :]
    base = jnp.cumsum(counts_chunk, axis=0) - counts_chunk
    counts = counts_chunk.sum(axis=0).astype(jnp.int32)
    within_sel = jnp.take_along_axis(within, cfm[:, :, None], axis=2)[:, :, 0]
    base_sel = jnp.take_along_axis(base, cfm, axis=1)
    ranks = (base_sel + within_sel - 1.0).astype(jnp.int32).reshape(-1)
    blocks = (counts + _BM - 1) // _BM
    cumblocks = jnp.cumsum(blocks)
    used = cumblocks[-1]
    padoff = (jnp.concatenate([jnp.zeros((1,), cumblocks.dtype), cumblocks[:-1]]) * _BM)
    pos = (padoff[cf] + ranks).astype(jnp.int32)
    tok = jnp.arange(_ROWS, dtype=jnp.int32) // _K
    rowtok = jnp.zeros((_MAX_BLOCKS * _BM,), jnp.int32).at[pos].set(tok)
    wrow = jnp.zeros((_MAX_BLOCKS * _BM,), jnp.float32).at[pos].set(cw.reshape(-1))

    bidx = jnp.arange(_MAX_BLOCKS, dtype=jnp.int32)
    be = jnp.minimum(jnp.searchsorted(cumblocks, bidx, side="right"), _E - 1).astype(jnp.int32)
    valid = (bidx < used)
    last = (used - 1).astype(jnp.int32)
    be = jnp.where(valid, be, be[last])
    xm = jnp.where(valid, bidx, last)
    vd = valid.astype(jnp.int32)

    sorted_x = _sc_gather_rows(xf, rowtok, _MAX_BLOCKS * _BM)
    eo = _grouped_ffn(sorted_x, gating_einsum, linear,
                      wrow.reshape(_MAX_BLOCKS, 1, _BM), be, xm, vd)
    tm = _sc_gather_rows(eo, pos, _ROWS)
    out = _sum8(tm)
    return out.reshape(g, s, d)


# speedup vs baseline: 1.0009x; 1.0009x over previous
import functools

import jax
import jax.numpy as jnp
from jax.experimental import pallas as pl
from jax.experimental.pallas import tpu as pltpu
from jax.experimental.pallas import tpu_sc as plsc

_EMBED = 1024
_HIDDEN = 512
_E = 64
_K = 8
_T = 2048
_BM = 256
_ROWS = _T * _K
_MAX_BLOCKS = _ROWS // _BM + _E
_NW = 32
_GCH = 64
_CHT = 8


def _sc_mesh():
    return plsc.VectorSubcoreMesh(core_axis_name="c", subcore_axis_name="s")


def _sc_gather_rows(table, idx, n_rows):
    per_w = n_rows // _NW
    nch = per_w // _GCH

    @functools.partial(
        pl.kernel,
        out_type=jax.ShapeDtypeStruct((n_rows, _EMBED), jnp.float32),
        mesh=_sc_mesh(),
        scratch_types=[
            pltpu.VMEM((_GCH,), jnp.int32),
            pltpu.VMEM((_GCH, _EMBED), jnp.float32),
            pltpu.SemaphoreType.DMA,
        ],
    )
    def k(tab_hbm, idx_hbm, out_hbm, idx_v, rows_v, sem):
        wid = jax.lax.axis_index("s") * 2 + jax.lax.axis_index("c")
        base = wid * per_w

        @pl.loop(0, nch)
        def _(c):
            b = base + c * _GCH
            pltpu.sync_copy(idx_hbm.at[pl.ds(b, _GCH)], idx_v)
            pltpu.async_copy(tab_hbm.at[idx_v], rows_v, sem).wait()
            pltpu.sync_copy(rows_v, out_hbm.at[pl.ds(b, _GCH)])

    return k(table, idx)


def _sum8_kernel(x_ref, o_ref):
    o_ref[...] = x_ref[...].reshape(_BM, _K, _EMBED).sum(axis=1)


def _sum8(tm):
    return pl.pallas_call(
        _sum8_kernel,
        grid=(_T // _BM,),
        in_specs=[pl.BlockSpec((_BM * _K, _EMBED), lambda i: (i, 0))],
        out_specs=pl.BlockSpec((_BM, _EMBED), lambda i: (i, 0)),
        out_shape=jax.ShapeDtypeStruct((_T, _EMBED), jnp.float32),
    )(tm)


def _ffn_block_kernel(be_ref, xm_ref, vd_ref, x_ref, ge_ref, lin_ref, w_ref, out_ref):
    i = pl.program_id(0)

    @pl.when(vd_ref[i] == 1)
    def _():
        xb = x_ref[...].astype(jnp.bfloat16)
        g0 = ge_ref[0, 0].astype(jnp.bfloat16)
        g1 = ge_ref[0, 1].astype(jnp.bfloat16)
        x1 = jax.lax.dot_general(xb, g0, (((1,), (1,)), ((), ())),
                                 preferred_element_type=jnp.float32)
        x2 = jax.lax.dot_general(xb, g1, (((1,), (1,)), ((), ())),
                                 preferred_element_type=jnp.float32)
        act = (jax.nn.gelu(x1) * x2).astype(jnp.bfloat16)
        lin = lin_ref[0].astype(jnp.bfloat16)
        out = jax.lax.dot_general(act, lin, (((1,), (0,)), ((), ())),
                                  preferred_element_type=jnp.float32)
        out_ref[...] = out * jnp.transpose(w_ref[0])


def _grouped_ffn(sorted_x, gating_einsum, linear, wrow, block_expert, block_xmap, block_valid):
    grid_spec = pltpu.PrefetchScalarGridSpec(
        num_scalar_prefetch=3,
        grid=(_MAX_BLOCKS,),
        in_specs=[
            pl.BlockSpec((_BM, _EMBED), lambda i, be, xm, vd: (xm[i], 0)),
            pl.BlockSpec((1, 2, _HIDDEN, _EMBED), lambda i, be, xm, vd: (be[i], 0, 0, 0)),
            pl.BlockSpec((1, _HIDDEN, _EMBED), lambda i, be, xm, vd: (be[i], 0, 0)),
            pl.BlockSpec((1, 1, _BM), lambda i, be, xm, vd: (xm[i], 0, 0)),
        ],
        out_specs=pl.BlockSpec((_BM, _EMBED), lambda i, be, xm, vd: (xm[i], 0)),
    )
    return pl.pallas_call(
        _ffn_block_kernel,
        grid_spec=grid_spec,
        out_shape=jax.ShapeDtypeStruct((_MAX_BLOCKS * _BM, _EMBED), jnp.float32),
        compiler_params=pltpu.CompilerParams(
            dimension_semantics=("arbitrary",)),
    )(block_expert, block_xmap, block_valid, sorted_x, gating_einsum, linear, wrow)


def kernel(x, router_logits, gating_einsum, linear, per_expert_scale, router_scale):
    g, s, d = x.shape
    t = g * s
    xf = x.reshape(t, d)

    var = jnp.mean(jnp.square(xf), axis=-1, keepdims=True)
    ri = xf * jax.lax.rsqrt(var + 1e-06)
    ri = ri * jax.lax.rsqrt(jnp.float32(d)) * router_scale
    logits = ri @ router_logits
    probs = jax.nn.softmax(logits, axis=-1)
    _, choices = jax.lax.approx_max_k(logits, k=_K)
    indicator = jax.nn.one_hot(choices, _E, dtype=probs.dtype).sum(axis=-2)
    renorm = jnp.sum(indicator * probs, axis=-1, keepdims=True)
    renorm = jnp.where(renorm > 0.0, renorm, 1.0)
    cw = jnp.take_along_axis(probs / renorm, choices, axis=-1)
    cw = cw * per_expert_scale[choices]

    cf = choices.reshape(-1)
    nch = 128
    cfm = cf.reshape(nch, _ROWS // nch)
    ohc = (cfm[:, :, None] == jnp.arange(_E)).astype(jnp.bfloat16)
    tril = jnp.tril(jnp.ones((_ROWS // nch, _ROWS // nch), jnp.bfloat16))
    within = jnp.einsum('ij,cje->cie', tril, ohc,
                        preferred_element_type=jnp.float32)
    counts_chunk = within[:, -1, :]
    base = jnp.cumsum(counts_chunk, axis=0) - counts_chunk
    counts = counts_chunk.sum(axis=0).astype(jnp.int32)
    within_sel = jnp.take_along_axis(within, cfm[:, :, None], axis=2)[:, :, 0]
    base_sel = jnp.take_along_axis(base, cfm, axis=1)
    ranks = (base_sel + within_sel - 1.0).astype(jnp.int32).reshape(-1)
    blocks = (counts + _BM - 1) // _BM
    cumblocks = jnp.cumsum(blocks)
    used = cumblocks[-1]
    padoff = (jnp.concatenate([jnp.zeros((1,), cumblocks.dtype), cumblocks[:-1]]) * _BM)
    pos = (padoff[cf] + ranks).astype(jnp.int32)
    tok = jnp.arange(_ROWS, dtype=jnp.int32) // _K
    rowtok = jnp.zeros((_MAX_BLOCKS * _BM,), jnp.int32).at[pos].set(tok)
    wrow = jnp.zeros((_MAX_BLOCKS * _BM,), jnp.float32).at[pos].set(cw.reshape(-1))

    bidx = jnp.arange(_MAX_BLOCKS, dtype=jnp.int32)
    be = jnp.minimum(jnp.searchsorted(cumblocks, bidx, side="right"), _E - 1).astype(jnp.int32)
    valid = (bidx < used)
    last = (used - 1).astype(jnp.int32)
    be = jnp.where(valid, be, be[last])
    xm = jnp.where(valid, bidx, last)
    vd = valid.astype(jnp.int32)

    sorted_x = _sc_gather_rows(xf, rowtok, _MAX_BLOCKS * _BM)
    eo = _grouped_ffn(sorted_x, gating_einsum, linear,
                      wrow.reshape(_MAX_BLOCKS, 1, _BM), be, xm, vd)
    tm = _sc_gather_rows(eo, pos, _ROWS)
    out = _sum8(tm)
    return out.reshape(g, s, d)


# speedup vs baseline: 2.3462x; 2.3440x over previous
import functools

import jax
import jax.numpy as jnp
from jax.experimental import pallas as pl
from jax.experimental.pallas import tpu as pltpu
from jax.experimental.pallas import tpu_sc as plsc

_EMBED = 1024
_HIDDEN = 512
_E = 64
_K = 8
_T = 2048
_BM = 256
_ROWS = _T * _K
_MAX_BLOCKS = _ROWS // _BM + _E
_NW = 32
_GCH = 64
_CHT = 8


def _sc_mesh():
    return plsc.VectorSubcoreMesh(core_axis_name="c", subcore_axis_name="s")


def _sc_gather_rows(table, idx, n_rows):
    per_w = n_rows // _NW
    nch = per_w // _GCH

    @functools.partial(
        pl.kernel,
        out_type=jax.ShapeDtypeStruct((n_rows, _EMBED), jnp.float32),
        mesh=_sc_mesh(),
        scratch_types=[
            pltpu.VMEM((_GCH,), jnp.int32),
            pltpu.VMEM((_GCH, _EMBED), jnp.float32),
            pltpu.SemaphoreType.DMA,
        ],
    )
    def k(tab_hbm, idx_hbm, out_hbm, idx_v, rows_v, sem):
        wid = jax.lax.axis_index("s") * 2 + jax.lax.axis_index("c")
        base = wid * per_w

        @pl.loop(0, nch)
        def _(c):
            b = base + c * _GCH
            pltpu.sync_copy(idx_hbm.at[pl.ds(b, _GCH)], idx_v)
            pltpu.async_copy(tab_hbm.at[idx_v], rows_v, sem).wait()
            pltpu.sync_copy(rows_v, out_hbm.at[pl.ds(b, _GCH)])

    return k(table, idx)


def _copy_kernel(x_ref, o_ref):
    o_ref[...] = x_ref[...]


def _tc_copy(xf):
    return pl.pallas_call(
        _copy_kernel,
        grid=(_T // _BM,),
        in_specs=[pl.BlockSpec((_BM, _EMBED), lambda i: (i, 0))],
        out_specs=pl.BlockSpec((_BM, _EMBED), lambda i: (i, 0)),
        out_shape=jax.ShapeDtypeStruct((_T, _EMBED), jnp.float32),
    )(xf)


def _sum8_kernel(x_ref, o_ref):
    o_ref[...] = x_ref[...].reshape(_BM, _K, _EMBED).sum(axis=1)


def _sum8(tm):
    return pl.pallas_call(
        _sum8_kernel,
        grid=(_T // _BM,),
        in_specs=[pl.BlockSpec((_BM * _K, _EMBED), lambda i: (i, 0))],
        out_specs=pl.BlockSpec((_BM, _EMBED), lambda i: (i, 0)),
        out_shape=jax.ShapeDtypeStruct((_T, _EMBED), jnp.float32),
    )(tm)


def _ffn_block_kernel(be_ref, xm_ref, vd_ref, x_ref, ge_ref, lin_ref, w_ref, out_ref):
    i = pl.program_id(0)

    @pl.when(vd_ref[i] == 1)
    def _():
        xb = x_ref[...].astype(jnp.bfloat16)
        g0 = ge_ref[0, 0].astype(jnp.bfloat16)
        g1 = ge_ref[0, 1].astype(jnp.bfloat16)
        x1 = jax.lax.dot_general(xb, g0, (((1,), (1,)), ((), ())),
                                 preferred_element_type=jnp.float32)
        x2 = jax.lax.dot_general(xb, g1, (((1,), (1,)), ((), ())),
                                 preferred_element_type=jnp.float32)
        act = (jax.nn.gelu(x1) * x2).astype(jnp.bfloat16)
        lin = lin_ref[0].astype(jnp.bfloat16)
        out = jax.lax.dot_general(act, lin, (((1,), (0,)), ((), ())),
                                  preferred_element_type=jnp.float32)
        out_ref[...] = out * jnp.transpose(w_ref[0])


def _grouped_ffn(sorted_x, gating_einsum, linear, wrow, block_expert, block_xmap, block_valid):
    grid_spec = pltpu.PrefetchScalarGridSpec(
        num_scalar_prefetch=3,
        grid=(_MAX_BLOCKS,),
        in_specs=[
            pl.BlockSpec((_BM, _EMBED), lambda i, be, xm, vd: (xm[i], 0)),
            pl.BlockSpec((1, 2, _HIDDEN, _EMBED), lambda i, be, xm, vd: (be[i], 0, 0, 0)),
            pl.BlockSpec((1, _HIDDEN, _EMBED), lambda i, be, xm, vd: (be[i], 0, 0)),
            pl.BlockSpec((1, 1, _BM), lambda i, be, xm, vd: (xm[i], 0, 0)),
        ],
        out_specs=pl.BlockSpec((_BM, _EMBED), lambda i, be, xm, vd: (xm[i], 0)),
    )
    return pl.pallas_call(
        _ffn_block_kernel,
        grid_spec=grid_spec,
        out_shape=jax.ShapeDtypeStruct((_MAX_BLOCKS * _BM, _EMBED), jnp.float32),
        compiler_params=pltpu.CompilerParams(
            dimension_semantics=("arbitrary",)),
    )(block_expert, block_xmap, block_valid, sorted_x, gating_einsum, linear, wrow)


def kernel(x, router_logits, gating_einsum, linear, per_expert_scale, router_scale):
    g, s, d = x.shape
    t = g * s
    xf = x.reshape(t, d)

    var = jnp.mean(jnp.square(xf), axis=-1, keepdims=True)
    ri = xf * jax.lax.rsqrt(var + 1e-06)
    ri = ri * jax.lax.rsqrt(jnp.float32(d)) * router_scale
    logits = ri @ router_logits
    probs = jax.nn.softmax(logits, axis=-1)
    _, choices = jax.lax.approx_max_k(logits, k=_K)
    indicator = jax.nn.one_hot(choices, _E, dtype=probs.dtype).sum(axis=-2)
    renorm = jnp.sum(indicator * probs, axis=-1, keepdims=True)
    renorm = jnp.where(renorm > 0.0, renorm, 1.0)
    cw = jnp.take_along_axis(probs / renorm, choices, axis=-1)
    cw = cw * per_expert_scale[choices]

    cf = choices.reshape(-1)
    ohi = jax.nn.one_hot(cf, _E, dtype=jnp.int32)
    counts = ohi.sum(axis=0)
    ranks = jnp.take_along_axis(jnp.cumsum(ohi, axis=0), cf[:, None], axis=1)[:, 0] - 1
    blocks = (counts + _BM - 1) // _BM
    cumblocks = jnp.cumsum(blocks)
    used = cumblocks[-1]
    padoff = (jnp.concatenate([jnp.zeros((1,), cumblocks.dtype), cumblocks[:-1]]) * _BM)
    pos = (padoff[cf] + ranks).astype(jnp.int32)
    tok = jnp.arange(_ROWS, dtype=jnp.int32) // _K
    pad_ids = jnp.arange(_MAX_BLOCKS * _BM, dtype=jnp.int32) % _T
    rowtok = pad_ids.at[pos].set(tok)
    wrow = jnp.zeros((_MAX_BLOCKS * _BM,), jnp.float32).at[pos].set(cw.reshape(-1))

    bidx = jnp.arange(_MAX_BLOCKS, dtype=jnp.int32)
    be = jnp.minimum(jnp.searchsorted(cumblocks, bidx, side="right"), _E - 1).astype(jnp.int32)
    valid = (bidx < used)
    last = (used - 1).astype(jnp.int32)
    be = jnp.where(valid, be, be[last])
    xm = jnp.where(valid, bidx, last)
    vd = valid.astype(jnp.int32)

    sorted_x = _sc_gather_rows(_tc_copy(xf), rowtok, _MAX_BLOCKS * _BM)
    eo = _grouped_ffn(sorted_x, gating_einsum, linear,
                      wrow.reshape(_MAX_BLOCKS, 1, _BM), be, xm, vd)
    tm = _sc_gather_rows(eo, pos, _ROWS)
    out = _sum8(tm)
    return out.reshape(g, s, d)
